# baseline (speedup 1.0000x reference)
import functools

import jax
import jax.numpy as jnp
from jax import lax
from jax.experimental import pallas as pl
from jax.experimental.pallas import tpu as pltpu
from jax.experimental.pallas import tpu_sc as plsc

C = 1024
K = 8192
NUM_OPS = 64
NUM_CLASSES = 1000
BASELINE_DECAY = 0.99

NC = 2
NS = 16
NW = NC * NS
EPW = K // NW
CB = 128
NB = C // CB
LANES = 16


def _bfly(v, op):
    iota = lax.iota(jnp.int32, LANES)
    for d in (1, 2, 4, 8):
        v = op(v, v.at[iota ^ d].get(mode="promise_in_bounds",
                                     unique_indices=True))
    return v


def _body(an_hbm, ar_hbm, ea_hbm, idxn_hbm, idxr_hbm, outn_hbm, outr_hbm,
          ea_v, rew_v, idx_b, acc_v, row_v, sem0, sem1, sem2):
    wid = lax.axis_index("s") * NC + lax.axis_index("c")
    e0 = wid * EPW

    pltpu.sync_copy(ea_hbm, ea_v)

    def _sum_step(i, s):
        return s + ea_v[pl.ds(i * LANES, LANES)]
    tot = _bfly(lax.fori_loop(0, C // LANES, _sum_step,
                              jnp.zeros((LANES,), jnp.float32)), jnp.add)
    base0 = jnp.float32(1.0 / NUM_CLASSES)
    baseline = base0 + jnp.float32(BASELINE_DECAY) * (tot * jnp.float32(1.0 / C) - base0)
    rew_sum = tot - jnp.float32(C) * baseline

    def _rew_step(i, _):
        rew_v[pl.ds(i * LANES, LANES)] = ea_v[pl.ds(i * LANES, LANES)] - baseline
        return 0
    lax.fori_loop(0, C // LANES, _rew_step, 0)

    iota = lax.iota(jnp.int32, LANES)
    offs = [iota * NUM_OPS + j * (LANES * NUM_OPS) for j in range(EPW // LANES)]
    zero16 = jnp.zeros((LANES,), jnp.float32)

    for a_hbm, idx_hbm, out_hbm in ((an_hbm, idxn_hbm, outn_hbm),
                                    (ar_hbm, idxr_hbm, outr_hbm)):
        a_cp = pltpu.async_copy(a_hbm.at[pl.ds(e0, EPW)], row_v, sem2)

        def _start(b, buf):
            src = idx_hbm.at[pl.ds(b * CB, CB), pl.ds(e0, EPW)]
            if buf == 0:
                pltpu.async_copy(src, idx_b.at[0], sem0)
            else:
                pltpu.async_copy(src, idx_b.at[1], sem1)

        def _wait(buf):
            if buf == 0:
                pltpu.make_async_copy(idx_hbm.at[pl.ds(0, CB), pl.ds(e0, EPW)],
                                      idx_b.at[0], sem0).wait()
            else:
                pltpu.make_async_copy(idx_hbm.at[pl.ds(0, CB), pl.ds(e0, EPW)],
                                      idx_b.at[1], sem1).wait()

        _start(0, 0)

        @plsc.parallel_loop(0, EPW * NUM_OPS // LANES, 1, unroll=8)
        def _zero_step(i):
            acc_v[pl.ds(i * LANES, LANES)] = zero16

        def _block(b, _):
            cur = lax.rem(b, 2)

            @pl.when(b + 1 < NB)
            def _prefetch():
                @pl.when(cur == 0)
                def _p0():
                    _start(b + 1, 1)

                @pl.when(cur == 1)
                def _p1():
                    _start(b + 1, 0)

            def _clients(buf):
                @plsc.parallel_loop(0, CB, 1, unroll=8)
                def _client(ci):
                    rv = plsc.load_gather(
                        rew_v, [jnp.full((LANES,), b * CB + ci, jnp.int32)])
                    for j in range(EPW // LANES):
                        iv = idx_b[buf, ci, pl.ds(j * LANES, LANES)]
                        plsc.addupdate_scatter(acc_v, [iv + offs[j]], rv)

            @pl.when(cur == 0)
            def _c0():
                _wait(0)
                _clients(0)

            @pl.when(cur == 1)
            def _c1():
                _wait(1)
                _clients(1)
            return 0
        with jax.named_scope("scatter_phase"):
            lax.fori_loop(0, NB, _block, 0)

        a_cp.wait()

        with jax.named_scope("softmax_phase"):
            @plsc.parallel_loop(0, EPW, 1, unroll=4)
            def _row(e):
                xs = [row_v[e, pl.ds(i * LANES, LANES)] for i in range(NUM_OPS // LANES)]
                es = [jnp.exp(x) for x in xs]
                ssum = _bfly((es[0] + es[1]) + (es[2] + es[3]), jnp.add)
                k1 = rew_sum / (jnp.float32(C) * ssum)
                for i in range(NUM_OPS // LANES):
                    a = acc_v[pl.ds(e * NUM_OPS + i * LANES, LANES)]
                    row_v[e, pl.ds(i * LANES, LANES)] = es[i] * k1 - a * jnp.float32(1.0 / C)

        pltpu.sync_copy(row_v, out_hbm.at[pl.ds(e0, EPW)])


@jax.jit
def _grad_kernel(an, ar, ea, idxn, idxr):
    f = pl.kernel(
        _body,
        out_type=(jax.ShapeDtypeStruct((K, NUM_OPS), jnp.float32),
                  jax.ShapeDtypeStruct((K, NUM_OPS), jnp.float32)),
        mesh=plsc.VectorSubcoreMesh(core_axis_name="c", subcore_axis_name="s"),
        scratch_types=[
            pltpu.VMEM((C,), jnp.float32),
            pltpu.VMEM((C,), jnp.float32),
            pltpu.VMEM((2, CB, EPW), jnp.int32),
            pltpu.VMEM((EPW * NUM_OPS,), jnp.float32),
            pltpu.VMEM((EPW, NUM_OPS), jnp.float32),
            pltpu.SemaphoreType.DMA,
            pltpu.SemaphoreType.DMA,
            pltpu.SemaphoreType.DMA,
        ],
        compiler_params=pltpu.CompilerParams(needs_layout_passes=False),
    )
    return f(an, ar, ea, idxn, idxr)


def kernel(alphas_normal, alphas_reduce, epoch_acc, epoch_index_normal, epoch_index_reduce):
    idxn = epoch_index_normal.astype(jnp.int32)
    idxr = epoch_index_reduce.astype(jnp.int32)
    return _grad_kernel(alphas_normal, alphas_reduce, epoch_acc, idxn, idxr)


# speedup vs baseline: 1.0239x; 1.0239x over previous
import functools

import jax
import jax.numpy as jnp
from jax import lax
from jax.experimental import pallas as pl
from jax.experimental.pallas import tpu as pltpu
from jax.experimental.pallas import tpu_sc as plsc

C = 1024
K = 8192
NUM_OPS = 64
NUM_CLASSES = 1000
BASELINE_DECAY = 0.99

NC = 2
NS = 16
NW = NC * NS
EPW = K // NW
CB = 64
NB = C // CB
LANES = 16


def _bfly(v, op):
    iota = lax.iota(jnp.int32, LANES)
    for d in (1, 2, 4, 8):
        v = op(v, v.at[iota ^ d].get(mode="promise_in_bounds",
                                     unique_indices=True))
    return v


def _body(an_hbm, ar_hbm, ea_hbm, idxn_hbm, idxr_hbm, outn_hbm, outr_hbm,
          ea_v, rew_v, idx_b, acc_v, row_v, sem0, sem1, sem2):
    wid = lax.axis_index("s") * NC + lax.axis_index("c")
    e0 = wid * EPW

    pltpu.sync_copy(ea_hbm, ea_v)

    def _sum_step(i, s):
        return s + ea_v[pl.ds(i * LANES, LANES)]
    tot = _bfly(lax.fori_loop(0, C // LANES, _sum_step,
                              jnp.zeros((LANES,), jnp.float32)), jnp.add)
    base0 = jnp.float32(1.0 / NUM_CLASSES)
    baseline = base0 + jnp.float32(BASELINE_DECAY) * (tot * jnp.float32(1.0 / C) - base0)
    rew_sum = tot - jnp.float32(C) * baseline

    def _rew_step(i, _):
        rew_v[pl.ds(i * LANES, LANES)] = ea_v[pl.ds(i * LANES, LANES)] - baseline
        return 0
    lax.fori_loop(0, C // LANES, _rew_step, 0)

    iota = lax.iota(jnp.int32, LANES)
    offs = [iota * NUM_OPS + j * (LANES * NUM_OPS) for j in range(EPW // LANES)]
    zero16 = jnp.zeros((LANES,), jnp.float32)

    for a_hbm, idx_hbm, out_hbm in ((an_hbm, idxn_hbm, outn_hbm),
                                    (ar_hbm, idxr_hbm, outr_hbm)):
        a_cp = pltpu.async_copy(a_hbm.at[pl.ds(e0, EPW)], row_v, sem2)

        def _start(b, buf):
            src = idx_hbm.at[pl.ds(b * CB, CB), pl.ds(e0, EPW)]
            if buf == 0:
                pltpu.async_copy(src, idx_b.at[0], sem0)
            else:
                pltpu.async_copy(src, idx_b.at[1], sem1)

        def _wait(buf):
            if buf == 0:
                pltpu.make_async_copy(idx_hbm.at[pl.ds(0, CB), pl.ds(e0, EPW)],
                                      idx_b.at[0], sem0).wait()
            else:
                pltpu.make_async_copy(idx_hbm.at[pl.ds(0, CB), pl.ds(e0, EPW)],
                                      idx_b.at[1], sem1).wait()

        _start(0, 0)

        @plsc.parallel_loop(0, EPW * NUM_OPS // LANES, 1, unroll=8)
        def _zero_step(i):
            acc_v[pl.ds(i * LANES, LANES)] = zero16

        def _block(b, _):
            cur = lax.rem(b, 2)

            @pl.when(b + 1 < NB)
            def _prefetch():
                @pl.when(cur == 0)
                def _p0():
                    _start(b + 1, 1)

                @pl.when(cur == 1)
                def _p1():
                    _start(b + 1, 0)

            def _clients(buf):
                @plsc.parallel_loop(0, CB, 1, unroll=8)
                def _client(ci):
                    rv = plsc.load_gather(
                        rew_v, [jnp.full((LANES,), b * CB + ci, jnp.int32)])
                    for j in range(EPW // LANES):
                        iv = idx_b[buf, ci, pl.ds(j * LANES, LANES)]
                        plsc.addupdate_scatter(acc_v, [iv + offs[j]], rv)

            @pl.when(cur == 0)
            def _c0():
                _wait(0)
                _clients(0)

            @pl.when(cur == 1)
            def _c1():
                _wait(1)
                _clients(1)
            return 0
        with jax.named_scope("scatter_phase"):
            lax.fori_loop(0, NB, _block, 0)

        a_cp.wait()

        with jax.named_scope("softmax_phase"):
            @plsc.parallel_loop(0, EPW, 1, unroll=4)
            def _row(e):
                xs = [row_v[e, pl.ds(i * LANES, LANES)] for i in range(NUM_OPS // LANES)]
                es = [jnp.exp(x) for x in xs]
                ssum = _bfly((es[0] + es[1]) + (es[2] + es[3]), jnp.add)
                k1 = rew_sum / (jnp.float32(C) * ssum)
                for i in range(NUM_OPS // LANES):
                    a = acc_v[pl.ds(e * NUM_OPS + i * LANES, LANES)]
                    row_v[e, pl.ds(i * LANES, LANES)] = es[i] * k1 - a * jnp.float32(1.0 / C)

        pltpu.sync_copy(row_v, out_hbm.at[pl.ds(e0, EPW)])


@jax.jit
def _grad_kernel(an, ar, ea, idxn, idxr):
    f = pl.kernel(
        _body,
        out_type=(jax.ShapeDtypeStruct((K, NUM_OPS), jnp.float32),
                  jax.ShapeDtypeStruct((K, NUM_OPS), jnp.float32)),
        mesh=plsc.VectorSubcoreMesh(core_axis_name="c", subcore_axis_name="s"),
        scratch_types=[
            pltpu.VMEM((C,), jnp.float32),
            pltpu.VMEM((C,), jnp.float32),
            pltpu.VMEM((2, CB, EPW), jnp.int32),
            pltpu.VMEM((EPW * NUM_OPS,), jnp.float32),
            pltpu.VMEM((EPW, NUM_OPS), jnp.float32),
            pltpu.SemaphoreType.DMA,
            pltpu.SemaphoreType.DMA,
            pltpu.SemaphoreType.DMA,
        ],
        compiler_params=pltpu.CompilerParams(needs_layout_passes=False),
    )
    return f(an, ar, ea, idxn, idxr)


def kernel(alphas_normal, alphas_reduce, epoch_acc, epoch_index_normal, epoch_index_reduce):
    idxn = epoch_index_normal.astype(jnp.int32)
    idxr = epoch_index_reduce.astype(jnp.int32)
    return _grad_kernel(alphas_normal, alphas_reduce, epoch_acc, idxn, idxr)


# speedup vs baseline: 1.0243x; 1.0004x over previous
import jax
import jax.numpy as jnp
from jax import lax
from jax.experimental import pallas as pl
from jax.experimental.pallas import tpu as pltpu
from jax.experimental.pallas import tpu_sc as plsc

C = 1024
K = 8192
NUM_OPS = 64
NUM_CLASSES = 1000
BASELINE_DECAY = 0.99

NC = 2
NS = 16
NW = NC * NS
EPW = K // NW
CB = 64
NB = C // CB
LANES = 16


def _bfly(v, op):
    iota = lax.iota(jnp.int32, LANES)
    for d in (1, 2, 4, 8):
        v = op(v, v.at[iota ^ d].get(mode="promise_in_bounds",
                                     unique_indices=True))
    return v


def _body(an_hbm, ar_hbm, ea_hbm, idxn_hbm, idxr_hbm, outn_hbm, outr_hbm,
          ea_v, rew_v, idx_b, acc_v, row_v, sem0, sem1, sem2):
    wid = lax.axis_index("s") * NC + lax.axis_index("c")
    e0 = wid * EPW

    pltpu.sync_copy(ea_hbm, ea_v)

    def _sum_step(i, s):
        return s + ea_v[pl.ds(i * LANES, LANES)]
    tot = _bfly(lax.fori_loop(0, C // LANES, _sum_step,
                              jnp.zeros((LANES,), jnp.float32)), jnp.add)
    base0 = jnp.float32(1.0 / NUM_CLASSES)
    baseline = base0 + jnp.float32(BASELINE_DECAY) * (tot * jnp.float32(1.0 / C) - base0)
    rew_sum = tot - jnp.float32(C) * baseline

    def _rew_step(i, _):
        rew_v[pl.ds(i * LANES, LANES)] = ea_v[pl.ds(i * LANES, LANES)] - baseline
        return 0
    lax.fori_loop(0, C // LANES, _rew_step, 0)

    iota = lax.iota(jnp.int32, LANES)
    offs = [iota * NUM_OPS + j * (LANES * NUM_OPS) for j in range(EPW // LANES)]
    zero16 = jnp.zeros((LANES,), jnp.float32)

    for a_hbm, idx_hbm, out_hbm in ((an_hbm, idxn_hbm, outn_hbm),
                                    (ar_hbm, idxr_hbm, outr_hbm)):
        a_cp = pltpu.async_copy(a_hbm.at[pl.ds(e0, EPW)], row_v, sem2)

        def _start(b, buf):
            src = idx_hbm.at[pl.ds(b * CB, CB), pl.ds(e0, EPW)]
            if buf == 0:
                pltpu.async_copy(src, idx_b.at[0], sem0)
            else:
                pltpu.async_copy(src, idx_b.at[1], sem1)

        def _wait(buf):
            if buf == 0:
                pltpu.make_async_copy(idx_hbm.at[pl.ds(0, CB), pl.ds(e0, EPW)],
                                      idx_b.at[0], sem0).wait()
            else:
                pltpu.make_async_copy(idx_hbm.at[pl.ds(0, CB), pl.ds(e0, EPW)],
                                      idx_b.at[1], sem1).wait()

        _start(0, 0)

        @plsc.parallel_loop(0, EPW * NUM_OPS // LANES, 1, unroll=8)
        def _zero_step(i):
            acc_v[pl.ds(i * LANES, LANES)] = zero16

        def _block(b, _):
            cur = lax.rem(b, 2)

            @pl.when(b + 1 < NB)
            def _prefetch():
                @pl.when(cur == 0)
                def _p0():
                    _start(b + 1, 1)

                @pl.when(cur == 1)
                def _p1():
                    _start(b + 1, 0)

            def _clients(buf):
                @plsc.parallel_loop(0, CB, 1, unroll=8)
                def _client(ci):
                    rv = plsc.load_gather(
                        rew_v, [jnp.full((LANES,), b * CB + ci, jnp.int32)])
                    for j in range(EPW // LANES):
                        iv = idx_b[buf, ci, pl.ds(j * LANES, LANES)]
                        plsc.addupdate_scatter(acc_v, [iv + offs[j]], rv)

            @pl.when(cur == 0)
            def _c0():
                _wait(0)
                _clients(0)

            @pl.when(cur == 1)
            def _c1():
                _wait(1)
                _clients(1)
            return 0
        with jax.named_scope("scatter_phase"):
            lax.fori_loop(0, NB, _block, 0)

        a_cp.wait()

        with jax.named_scope("softmax_phase"):
            @plsc.parallel_loop(0, EPW, 1, unroll=4)
            def _row(e):
                xs = [row_v[e, pl.ds(i * LANES, LANES)] for i in range(NUM_OPS // LANES)]
                es = [jnp.exp(x) for x in xs]
                ssum = _bfly((es[0] + es[1]) + (es[2] + es[3]), jnp.add)
                k1 = rew_sum / (jnp.float32(C) * ssum)
                for i in range(NUM_OPS // LANES):
                    a = acc_v[pl.ds(e * NUM_OPS + i * LANES, LANES)]
                    row_v[e, pl.ds(i * LANES, LANES)] = es[i] * k1 - a * jnp.float32(1.0 / C)

        pltpu.sync_copy(row_v, out_hbm.at[pl.ds(e0, EPW)])


@jax.jit
def _grad_kernel(an, ar, ea, idxn, idxr):
    f = pl.kernel(
        _body,
        out_type=(jax.ShapeDtypeStruct((K, NUM_OPS), jnp.float32),
                  jax.ShapeDtypeStruct((K, NUM_OPS), jnp.float32)),
        mesh=plsc.VectorSubcoreMesh(core_axis_name="c", subcore_axis_name="s"),
        scratch_types=[
            pltpu.VMEM((C,), jnp.float32),
            pltpu.VMEM((C,), jnp.float32),
            pltpu.VMEM((2, CB, EPW), jnp.int32),
            pltpu.VMEM((EPW * NUM_OPS,), jnp.float32),
            pltpu.VMEM((EPW, NUM_OPS), jnp.float32),
            pltpu.SemaphoreType.DMA,
            pltpu.SemaphoreType.DMA,
            pltpu.SemaphoreType.DMA,
        ],
        compiler_params=pltpu.CompilerParams(needs_layout_passes=False),
    )
    return f(an, ar, ea, idxn, idxr)


def kernel(alphas_normal, alphas_reduce, epoch_acc, epoch_index_normal, epoch_index_reduce):
    idxn = epoch_index_normal.astype(jnp.int32)
    idxr = epoch_index_reduce.astype(jnp.int32)
    return _grad_kernel(alphas_normal, alphas_reduce, epoch_acc, idxn, idxr)
